# baseline (speedup 1.0000x reference)
import functools

import jax
import jax.numpy as jnp
from jax import lax
from jax.experimental import pallas as pl
from jax.experimental.pallas import tpu as pltpu
from jax.experimental.pallas import tpu_sc as plsc

_TOK = 2048
_CH = 128


def _vq_body(z_ref, e_ref, en_ref, zn_ref, idx_ref):
    zb = z_ref[...]
    e = e_ref[...]
    en = en_ref[...]
    zn = zn_ref[...]
    em2 = e * -2.0
    cross2 = jax.lax.dot_general(em2, zb, (((1,), (1,)), ((), ())))
    s = (zn + en) + cross2
    idx = jnp.argmin(s, axis=0).astype(jnp.int32)
    idx_ref[0, 0, :] = idx


def _tc_indices(zf, embeddings, en, znt):
    n_tok, e_dim = zf.shape
    n_codes = embeddings.shape[0]
    grid = n_tok // _TOK
    idx = pl.pallas_call(
        _vq_body,
        grid=(grid,),
        in_specs=[
            pl.BlockSpec((_TOK, e_dim), lambda i: (i, 0)),
            pl.BlockSpec((n_codes, e_dim), lambda i: (0, 0)),
            pl.BlockSpec((n_codes, 1), lambda i: (0, 0)),
            pl.BlockSpec((1, _TOK), lambda i: (0, i)),
        ],
        out_specs=pl.BlockSpec((1, 1, _TOK), lambda i: (i, 0, 0)),
        out_shape=jax.ShapeDtypeStruct((grid, 1, _TOK), jnp.int32),
    )(zf, embeddings, en, znt)
    return idx.reshape(n_tok)


def _make_sc_gather(n_tok, n_codes, e_dim):
    info = plsc.get_sparse_core_info()
    nc, ns = info.num_cores, info.num_subcores
    nw = nc * ns
    bpw = n_tok // nw
    mesh = plsc.VectorSubcoreMesh(core_axis_name="c", subcore_axis_name="s")

    @functools.partial(
        pl.kernel,
        mesh=mesh,
        compiler_params=pltpu.CompilerParams(use_tc_tiling_on_sc=False),
        out_type=jax.ShapeDtypeStruct((n_tok, e_dim), jnp.float32),
        scratch_types=[
            pltpu.VMEM((bpw,), jnp.int32),
            pltpu.VMEM((bpw, e_dim), jnp.float32),
            pltpu.SemaphoreType.DMA,
        ],
    )
    def sc_gather(e_hbm, idx_hbm, out_hbm, idx_v, rows_v, sem):
        wid = lax.axis_index("s") * nc + lax.axis_index("c")
        base = wid * bpw
        pltpu.sync_copy(idx_hbm.at[pl.ds(base, bpw)], idx_v)
        copies = []
        for j in range(bpw // _CH):
            copies.append(
                pltpu.async_copy(e_hbm.at[idx_v.at[pl.ds(j * _CH, _CH)]],
                                 rows_v.at[pl.ds(j * _CH, _CH)], sem))
        for c in copies:
            c.wait()
        pltpu.sync_copy(rows_v, out_hbm.at[pl.ds(base, bpw)])

    return sc_gather


def kernel(z, embeddings):
    e_dim = z.shape[-1]
    zf = z.reshape(-1, e_dim)
    n_tok = zf.shape[0]
    n_codes = embeddings.shape[0]
    en = jnp.sum(embeddings ** 2, axis=1, keepdims=True)
    znt = jnp.sum(zf ** 2, axis=1, keepdims=True).T
    idx = _tc_indices(zf, embeddings, en, znt)
    zq = _make_sc_gather(n_tok, n_codes, e_dim)(embeddings, idx)
    return zq.reshape(z.shape), idx.reshape(z.shape[:-1])


# speedup vs baseline: 1.0122x; 1.0122x over previous
import functools

import jax
import jax.numpy as jnp
from jax import lax
from jax.experimental import pallas as pl
from jax.experimental.pallas import tpu as pltpu
from jax.experimental.pallas import tpu_sc as plsc

_TOK = 8192
_CH = 128


def _vq_body(z_ref, e_ref, en_ref, zn_ref, idx_ref):
    zb = z_ref[...]
    e = e_ref[...]
    en = en_ref[...]
    zn = zn_ref[...]
    em2 = e * -2.0
    cross2 = jax.lax.dot_general(em2, zb, (((1,), (1,)), ((), ())))
    s = (zn + en) + cross2
    idx = jnp.argmin(s, axis=0).astype(jnp.int32)
    idx_ref[0, 0, :] = idx


def _tc_indices(zf, embeddings, en, znt):
    n_tok, e_dim = zf.shape
    n_codes = embeddings.shape[0]
    grid = n_tok // _TOK
    idx = pl.pallas_call(
        _vq_body,
        grid=(grid,),
        in_specs=[
            pl.BlockSpec((_TOK, e_dim), lambda i: (i, 0)),
            pl.BlockSpec((n_codes, e_dim), lambda i: (0, 0)),
            pl.BlockSpec((n_codes, 1), lambda i: (0, 0)),
            pl.BlockSpec((1, _TOK), lambda i: (0, i)),
        ],
        out_specs=pl.BlockSpec((1, 1, _TOK), lambda i: (i, 0, 0)),
        out_shape=jax.ShapeDtypeStruct((grid, 1, _TOK), jnp.int32),
    )(zf, embeddings, en, znt)
    return idx.reshape(n_tok)


def _make_sc_gather(n_tok, n_codes, e_dim):
    info = plsc.get_sparse_core_info()
    nc, ns = info.num_cores, info.num_subcores
    nw = nc * ns
    bpw = n_tok // nw
    mesh = plsc.VectorSubcoreMesh(core_axis_name="c", subcore_axis_name="s")

    @functools.partial(
        pl.kernel,
        mesh=mesh,
        compiler_params=pltpu.CompilerParams(use_tc_tiling_on_sc=False),
        out_type=jax.ShapeDtypeStruct((n_tok, e_dim), jnp.float32),
        scratch_types=[
            pltpu.VMEM((bpw,), jnp.int32),
            pltpu.VMEM((bpw, e_dim), jnp.float32),
            pltpu.SemaphoreType.DMA,
        ],
    )
    def sc_gather(e_hbm, idx_hbm, out_hbm, idx_v, rows_v, sem):
        wid = lax.axis_index("s") * nc + lax.axis_index("c")
        base = wid * bpw
        pltpu.sync_copy(idx_hbm.at[pl.ds(base, bpw)], idx_v)
        copies = []
        for j in range(bpw // _CH):
            copies.append(
                pltpu.async_copy(e_hbm.at[idx_v.at[pl.ds(j * _CH, _CH)]],
                                 rows_v.at[pl.ds(j * _CH, _CH)], sem))
        for c in copies:
            c.wait()
        pltpu.sync_copy(rows_v, out_hbm.at[pl.ds(base, bpw)])

    return sc_gather


def kernel(z, embeddings):
    e_dim = z.shape[-1]
    zf = z.reshape(-1, e_dim)
    n_tok = zf.shape[0]
    n_codes = embeddings.shape[0]
    en = jnp.sum(embeddings ** 2, axis=1, keepdims=True)
    znt = jnp.sum(zf ** 2, axis=1, keepdims=True).T
    idx = _tc_indices(zf, embeddings, en, znt)
    zq = _make_sc_gather(n_tok, n_codes, e_dim)(embeddings, idx)
    return zq.reshape(z.shape), idx.reshape(z.shape[:-1])
